# Initial kernel scaffold; baseline (speedup 1.0000x reference)
import jax
import jax.numpy as jnp
from jax.experimental import pallas as pl


def kernel(input, weights):
    raise NotImplementedError("write your pallas kernel here")


# speedup vs baseline: 4.8782x; 4.8782x over previous
import functools

import jax
import jax.numpy as jnp
from jax import lax
from jax.experimental import pallas as pl
from jax.experimental.pallas import tpu as pltpu
from jax.experimental.pallas import tpu_sc as plsc

L = 16


def _build_sc_call(bsz, seq, dim, n_rows_w, nw, nc):
    sd = seq * dim
    tab_words = (seq + 1) * dim
    chunk = n_rows_w * seq
    n_full = seq // L
    rem = seq - n_full * L
    dpv = dim // L

    mesh = plsc.VectorSubcoreMesh(core_axis_name="c", subcore_axis_name="s")

    @functools.partial(
        pl.kernel,
        mesh=mesh,
        compiler_params=pltpu.CompilerParams(needs_layout_passes=False),
        out_type=jax.ShapeDtypeStruct((bsz * sd,), jnp.float32),
        scratch_types=[
            pltpu.VMEM((tab_words,), jnp.float32),
            pltpu.VMEM((chunk + L,), jnp.int32),
            pltpu.VMEM((sd,), jnp.float32),
            pltpu.VMEM((sd,), jnp.float32),
            pltpu.VMEM((L,), jnp.int32),
            pltpu.SemaphoreType.DMA,
            pltpu.SemaphoreType.DMA,
        ],
    )
    def sc_embed(inp_hbm, w_hbm, out_hbm, tab_v, inp_v, ob0, ob1, flag_v,
                 sem0, sem1):
        wid = lax.axis_index("s") * nc + lax.axis_index("c")
        base = wid * n_rows_w

        pltpu.sync_copy(w_hbm.at[pl.ds(0, tab_words)], tab_v)
        pltpu.sync_copy(inp_hbm.at[pl.ds(base * seq, chunk)],
                        inp_v.at[pl.ds(0, chunk)])

        zero_v = jnp.zeros((L,), jnp.int32)
        lane = lax.iota(jnp.int32, L)
        pad = [tab_v[pl.ds(j * L, L)] for j in range(dpv)]

        def row_has_pad(off):
            acc = jnp.zeros((L,), jnp.int32)
            for k in range(n_full):
                v = inp_v[pl.ds(off + k * L, L)]
                acc = jnp.bitwise_or(acc, (v == zero_v).astype(jnp.int32))
            if rem:
                v = inp_v[pl.ds(off + n_full * L, L)]
                acc = jnp.bitwise_or(
                    acc,
                    jnp.logical_and(v == zero_v, lane < rem).astype(jnp.int32))
            c = acc[0]
            for q in range(1, L):
                c = jnp.bitwise_or(c, acc[q])
            return c > 0

        def do_row(i, r, ob, sem):
            b = base + r
            off = r * seq

            @pl.when(i > 0)
            def _wait_prev():
                pltpu.make_async_copy(
                    ob, out_hbm.at[pl.ds(0, sd)], sem).wait()

            any_pad = row_has_pad(off)

            @pl.when(any_pad)
            def _slow():
                def s_body(s, c):
                    idx = jnp.full((L,), off + s, jnp.int32)
                    mv = plsc.load_gather(inp_v, [idx])
                    mb = mv != zero_v
                    for j in range(dpv):
                        t = tab_v[pl.ds((s + 1) * dim + j * L, L)]
                        ob[pl.ds(s * dim + j * L, L)] = jnp.where(
                            mb, t, pad[j])
                    return c
                lax.fori_loop(0, seq, s_body, 0)
                pltpu.async_copy(ob, out_hbm.at[pl.ds(b * sd, sd)], sem)

            @pl.when(jnp.logical_not(any_pad))
            def _fast():
                pltpu.async_copy(
                    tab_v.at[pl.ds(dim, sd)],
                    out_hbm.at[pl.ds(b * sd, sd)], sem)

        def pair_body(i, c):
            do_row(i, 2 * i, ob0, sem0)
            do_row(i, 2 * i + 1, ob1, sem1)
            return c

        lax.fori_loop(0, n_rows_w // 2, pair_body, 0)
        pltpu.make_async_copy(ob0, out_hbm.at[pl.ds(0, sd)], sem0).wait()
        pltpu.make_async_copy(ob1, out_hbm.at[pl.ds(0, sd)], sem1).wait()

    return sc_embed


def kernel(input, weights):
    bsz, seq = input.shape
    dim = weights.shape[1]
    info = plsc.get_sparse_core_info()
    nc, ns = info.num_cores, info.num_subcores
    nw = nc * ns
    n_rows_w = bsz // nw
    sc_embed = _build_sc_call(bsz, seq, dim, n_rows_w, nw, nc)
    out = sc_embed(input.reshape(-1), weights.reshape(-1))
    return out.reshape(bsz, seq, dim)


# speedup vs baseline: 19.3801x; 3.9728x over previous
import functools

import jax
import jax.numpy as jnp
from jax import lax
from jax.experimental import pallas as pl
from jax.experimental.pallas import tpu as pltpu
from jax.experimental.pallas import tpu_sc as plsc

L = 16


def _build_sc_call(bsz, seq, dim, bpw, nc):
    tab_words = (seq + 1) * dim
    chunk = bpw * seq
    dvec = dim // L
    bvec = bpw // L
    sg = 128 // dim
    ngrp = seq // sg

    mesh = plsc.VectorSubcoreMesh(core_axis_name="c", subcore_axis_name="s")

    @functools.partial(
        pl.kernel,
        mesh=mesh,
        compiler_params=pltpu.CompilerParams(needs_layout_passes=False),
        out_type=jax.ShapeDtypeStruct((seq * dim, bsz), jnp.float32),
        scratch_types=[
            pltpu.VMEM((tab_words,), jnp.float32),
            pltpu.VMEM((dim * L,), jnp.float32),
            pltpu.VMEM((chunk + L,), jnp.int32),
            pltpu.VMEM((chunk,), jnp.int32),
            pltpu.VMEM((sg * dim, bpw), jnp.float32),
            pltpu.VMEM((sg * dim, bpw), jnp.float32),
            pltpu.SemaphoreType.DMA,
            pltpu.SemaphoreType.DMA,
        ],
    )
    def sc_embed(inp_hbm, w_hbm, out_hbm, tab_v, pad_v, inp_v, inpt_v,
                 blk0, blk1, sem0, sem1):
        wid = lax.axis_index("s") * nc + lax.axis_index("c")
        base = wid * bpw

        pltpu.sync_copy(w_hbm.at[pl.ds(0, tab_words)], tab_v)
        pltpu.sync_copy(inp_hbm.at[pl.ds(base * seq, chunk)],
                        inp_v.at[pl.ds(0, chunk)])

        iota = lax.iota(jnp.int32, L)
        zero_v = jnp.zeros((L,), jnp.int32)

        def pad_body(d, c):
            p = plsc.load_gather(tab_v, [jnp.full((L,), d, jnp.int32)])
            pad_v[pl.ds(d * L, L)] = p
            return c
        lax.fori_loop(0, dim, pad_body, 0)

        iota_seq = iota * seq

        def tr_body(t, c):
            s = t // bvec
            j16 = t - s * bvec
            idx = iota_seq + (j16 * L * seq + s)
            v = plsc.load_gather(inp_v, [idx])
            inpt_v[pl.ds(s * bpw + j16 * L, L)] = v
            return c
        lax.fori_loop(0, seq * bvec, tr_body, 0, unroll=4)

        def do_spos(s, blk, row0):
            m = [inpt_v[pl.ds(s * bpw + j * L, L)] != zero_v
                 for j in range(bvec)]
            sbase = (s + 1) * dim

            def d_body(d, c):
                t = plsc.load_gather(
                    tab_v, [jnp.full((L,), sbase + d, jnp.int32)])
                p = pad_v[pl.ds(d * L, L)]
                r = row0 + d
                for j in range(bvec):
                    blk[r, pl.ds(j * L, L)] = jnp.where(m[j], t, p)
                return c
            lax.fori_loop(0, dim, d_body, 0, unroll=4)

        def do_group(g, blk, sem, primed):
            @pl.when(primed)
            def _wait_prev():
                pltpu.make_async_copy(
                    blk, out_hbm.at[pl.ds(0, sg * dim), pl.ds(base, bpw)],
                    sem).wait()

            for ss in range(sg):
                do_spos(g * sg + ss, blk, ss * dim)
            pltpu.async_copy(
                blk,
                out_hbm.at[pl.ds(g * sg * dim, sg * dim), pl.ds(base, bpw)],
                sem)

        def pair_body(i, c):
            do_group(2 * i, blk0, sem0, i > 0)
            do_group(2 * i + 1, blk1, sem1, i > 0)
            return c

        lax.fori_loop(0, ngrp // 2, pair_body, 0)
        pltpu.make_async_copy(
            blk0, out_hbm.at[pl.ds(0, sg * dim), pl.ds(base, bpw)],
            sem0).wait()
        pltpu.make_async_copy(
            blk1, out_hbm.at[pl.ds(0, sg * dim), pl.ds(base, bpw)],
            sem1).wait()

    return sc_embed


def kernel(input, weights):
    bsz, seq = input.shape
    dim = weights.shape[1]
    info = plsc.get_sparse_core_info()
    nc, ns = info.num_cores, info.num_subcores
    nw = nc * ns
    bpw = bsz // nw
    sc_embed = _build_sc_call(bsz, seq, dim, bpw, nc)
    out = sc_embed(input.reshape(-1), weights.reshape(-1))
    return out.reshape(seq, dim, bsz).transpose(2, 0, 1)
